# baseline (speedup 1.0000x reference)
import jax
import jax.numpy as jnp
from jax.experimental import pallas as pl
from jax.experimental.pallas import tpu as pltpu
from functools import partial

B, C, W, H = 32, 768, 32, 32
N = W * H
CB = 64
NBLK = C // CB
EPS = 0.001


def _weights_kernel(x_ref, wt_ref, s_acc):
    i = pl.program_id(0)

    @pl.when(i == 0)
    def _():
        s_acc[...] = jnp.zeros_like(s_acc)

    s_acc[...] += x_ref[...].sum(axis=1)

    @pl.when(i == NBLK - 1)
    def _():
        s = s_acc[...]
        mf = (s != 0).astype(jnp.float32)
        cnt = mf.sum(axis=1, keepdims=True)
        maxn = jnp.max(cnt)
        extra = maxn - cnt
        p0 = (jax.lax.broadcasted_iota(jnp.int32, (B, N), 1) == 0)
        wt = mf + jnp.where(p0, extra, 0.0)
        denom = jnp.float32(B) * maxn
        inv = jnp.where(denom > 0, 1.0 / denom, 0.0)
        wt_ref[...] = wt * inv


def _norm_kernel(x_ref, wt_ref, o_ref):
    xb = x_ref[...]
    wt = wt_ref[...]
    xw = xb * wt[:, None, :]
    mean = xw.sum(axis=(0, 2))
    ex2 = (xw * xb).sum(axis=(0, 2))
    scale = jax.lax.rsqrt(ex2 - mean * mean + EPS)
    write = wt > 0
    o_ref[...] = jnp.where(write[:, None, :], xb * scale[None, :, None], xb)


@jax.jit
def kernel(x):
    x3 = x.reshape(B, C, N)
    wt = pl.pallas_call(
        _weights_kernel,
        grid=(NBLK,),
        in_specs=[pl.BlockSpec((B, CB, N), lambda i: (0, i, 0))],
        out_specs=pl.BlockSpec((B, N), lambda i: (0, 0)),
        out_shape=jax.ShapeDtypeStruct((B, N), jnp.float32),
        scratch_shapes=[pltpu.VMEM((B, N), jnp.float32)],
    )(x3)
    out = pl.pallas_call(
        _norm_kernel,
        grid=(NBLK,),
        in_specs=[
            pl.BlockSpec((B, CB, N), lambda i: (0, i, 0)),
            pl.BlockSpec((B, N), lambda i: (0, 0)),
        ],
        out_specs=pl.BlockSpec((B, CB, N), lambda i: (0, i, 0)),
        out_shape=jax.ShapeDtypeStruct((B, C, N), jnp.float32),
    )(x3, wt)
    return out.reshape(B, C, W, H)
